# baseline (speedup 1.0000x reference)
import jax
import jax.numpy as jnp
from jax import lax
from jax.experimental import pallas as pl
from jax.experimental.pallas import tpu as pltpu
from jax.experimental.pallas import tpu_sc as plsc

CAP = 1000000
FEAT = 32
B = 16384
NC = 2
NS = 16
L = 16
NW = NC * NS
NA = 24
RPA = 31248
RPB = 31256
CW = 59904
NFULL = 16
TA = RPA * FEAT - NFULL * CW
TB = RPB * FEAT - NFULL * CW
NIDX = B // L
CAPL = 1024
LISTN = CAPL + 2 * L
PSHIFT = 16384


def _body(bxf, by, xf, y, idx, obxf, oby,
          llist, plist, wl, wp, pk, mbuf,
          sem_in0, sem_in1, sem_out0, sem_out1, sem_s):
  wid = lax.axis_index("s") * NC + lax.axis_index("c")
  base = wid * RPA + jnp.maximum(wid - NA, 0) * (RPB - RPA)
  is_b = wid >= NA
  rpw = jnp.where(is_b, RPB, RPA)
  iota = lax.iota(jnp.int32, L)
  zeros = jnp.zeros((L,), jnp.int32)

  def _phase1(u_buf, tab, by_buf):
    pltpu.sync_copy(idx, u_buf)

    def _zero(j, _):
      llist[pl.ds(j * L, L)] = zeros
      plist[pl.ds(j * L, L)] = zeros
      return 0
    lax.fori_loop(0, LISTN // L, _zero, 0)

    def _filter(k, cnt):
      v = u_buf[pl.ds(k * L, L)]
      inr = jnp.logical_and(v >= base, v < base + rpw)
      pos = k * L + iota
      inr_i = inr.astype(jnp.int32)
      cum = plsc.cumsum(inr_i)
      offs = cnt + cum - inr_i
      plsc.store_scatter(llist, [offs], v - base, mask=inr)
      plsc.store_scatter(plist, [offs], pos, mask=inr)
      return jnp.minimum(cnt + cum[L - 1], CAPL)
    n = lax.fori_loop(0, NIDX, _filter, jnp.int32(0))

    def _ded1(g, _):
      lanes = g * L + iota
      valid = lanes < n
      iv = llist[pl.ds(g * L, L)]
      pv = plist[pl.ds(g * L, L)]
      for l in range(L):
        plsc.store_scatter(tab, [iv], pv,
                           mask=jnp.logical_and(valid, iota == l))
      return 0
    lax.fori_loop(0, (n + L - 1) // L, _ded1, 0)

    def _ded2(g, m):
      lanes = g * L + iota
      valid = lanes < n
      iv = llist[pl.ds(g * L, L)]
      pv = plist[pl.ds(g * L, L)]
      w = plsc.load_gather(tab, [iv], mask=valid)
      win = jnp.logical_and(valid, w == pv)
      win_i = win.astype(jnp.int32)
      cum = plsc.cumsum(win_i)
      offs = m + cum - win_i
      plsc.store_scatter(wl, [offs], iv, mask=win)
      plsc.store_scatter(wp, [offs], pv, mask=win)
      return jnp.minimum(m + cum[L - 1], CAPL)
    m = lax.fori_loop(0, (n + L - 1) // L, _ded2, jnp.int32(0))

    mbuf[pl.ds(0, L)] = jnp.where(iota == 0, m, 0)

    @pl.when(jnp.logical_not(is_b))
    def _():
      pltpu.sync_copy(by.at[pl.ds(base, RPA)], by_buf.at[pl.ds(0, RPA)])

    @pl.when(is_b)
    def _():
      pltpu.sync_copy(by.at[pl.ds(base, RPB)], by_buf.at[pl.ds(0, RPB)])

    @pl.when(m > 0)
    def _():
      def _pack(g, _):
        lv = wl[pl.ds(g * L, L)]
        pv = wp[pl.ds(g * L, L)]
        pk[pl.ds(g * L, L)] = jnp.bitwise_or(lv * PSHIFT, pv)
        return 0
      lax.fori_loop(0, (m + L - 1) // L, _pack, 0)

      pltpu.sync_copy(y, u_buf)

      def _appy(g, _):
        lanes = g * L + iota
        msk = lanes < m
        iv = wl[pl.ds(g * L, L)]
        pv = wp[pl.ds(g * L, L)]
        yvv = plsc.load_gather(u_buf, [pv], mask=msk)
        plsc.store_scatter(by_buf, [iv], yvv, mask=msk)
        return 0
      lax.fori_loop(0, (m + L - 1) // L, _appy, 0)

    @pl.when(jnp.logical_not(is_b))
    def _():
      pltpu.sync_copy(by_buf.at[pl.ds(0, RPA)], oby.at[pl.ds(base, RPA)])

    @pl.when(is_b)
    def _():
      pltpu.sync_copy(by_buf.at[pl.ds(0, RPB)], oby.at[pl.ds(base, RPB)])

  pl.run_scoped(_phase1,
                pltpu.VMEM((B,), jnp.int32),
                pltpu.VMEM((RPB,), jnp.int32),
                pltpu.VMEM((RPB,), jnp.int32))

  m = mbuf[pl.ds(0, L)][0]

  fbase = base * FEAT

  def _phase2(buf0, buf1):
    def _wait_out0():
      pltpu.make_async_copy(buf0, obxf.at[pl.ds(fbase, CW)], sem_out0).wait()

    def _wait_out1():
      pltpu.make_async_copy(buf1, obxf.at[pl.ds(fbase, CW)], sem_out1).wait()

    def _pair(c2, _):
      o0 = fbase + (2 * c2) * CW
      o1 = o0 + CW

      @pl.when(c2 > 0)
      def _():
        _wait_out0()
      pltpu.async_copy(bxf.at[pl.ds(o0, CW)], buf0, sem_in0)

      @pl.when(c2 > 0)
      def _():
        _wait_out1()
      pltpu.async_copy(bxf.at[pl.ds(o1, CW)], buf1, sem_in1)

      pltpu.make_async_copy(bxf.at[pl.ds(o0, CW)], buf0, sem_in0).wait()
      pltpu.async_copy(buf0, obxf.at[pl.ds(o0, CW)], sem_out0)
      pltpu.make_async_copy(bxf.at[pl.ds(o1, CW)], buf1, sem_in1).wait()
      pltpu.async_copy(buf1, obxf.at[pl.ds(o1, CW)], sem_out1)
      return 0
    lax.fori_loop(0, NFULL // 2, _pair, 0)

    ot = fbase + NFULL * CW

    @pl.when(jnp.logical_not(is_b))
    def _():
      tsrc = bxf.at[pl.ds(ot, TA)]
      tdst = obxf.at[pl.ds(ot, TA)]
      tbuf = buf0.at[pl.ds(0, TA)]
      _wait_out0()
      pltpu.async_copy(tsrc, tbuf, sem_in0)
      pltpu.make_async_copy(tsrc, tbuf, sem_in0).wait()
      pltpu.async_copy(tbuf, tdst, sem_out0)
      pltpu.make_async_copy(tbuf, tdst, sem_out0).wait()

    @pl.when(is_b)
    def _():
      tsrc = bxf.at[pl.ds(ot, TB)]
      tdst = obxf.at[pl.ds(ot, TB)]
      tbuf = buf0.at[pl.ds(0, TB)]
      _wait_out0()
      pltpu.async_copy(tsrc, tbuf, sem_in0)
      pltpu.make_async_copy(tsrc, tbuf, sem_in0).wait()
      pltpu.async_copy(tbuf, tdst, sem_out0)
      pltpu.make_async_copy(tbuf, tdst, sem_out0).wait()

    _wait_out1()

  pl.run_scoped(_phase2,
                pltpu.VMEM((CW,), jnp.float32),
                pltpu.VMEM((CW,), jnp.float32))

  @pl.when(m > 0)
  def _():
    def _phase3(rowbuf):
      def _blk(g, _):
        vec = pk[pl.ds(g * L, L)]
        for k in range(L):
          @pl.when(g * L + k < m)
          def _():
            e = vec[k]
            p = jax.lax.rem(e, PSHIFT)
            slot = (g * L + k) * FEAT
            pltpu.async_copy(xf.at[pl.ds(p * FEAT, FEAT)],
                             rowbuf.at[pl.ds(slot, FEAT)], sem_s)
        for k in range(L):
          @pl.when(g * L + k < m)
          def _():
            e = vec[k]
            p = jax.lax.rem(e, PSHIFT)
            slot = (g * L + k) * FEAT
            pltpu.make_async_copy(xf.at[pl.ds(p * FEAT, FEAT)],
                                  rowbuf.at[pl.ds(slot, FEAT)], sem_s).wait()
        for k in range(L):
          @pl.when(g * L + k < m)
          def _():
            e = vec[k]
            r = jax.lax.div(e, PSHIFT)
            slot = (g * L + k) * FEAT
            pltpu.async_copy(rowbuf.at[pl.ds(slot, FEAT)],
                             obxf.at[pl.ds((base + r) * FEAT, FEAT)], sem_s)
        for k in range(L):
          @pl.when(g * L + k < m)
          def _():
            e = vec[k]
            r = jax.lax.div(e, PSHIFT)
            slot = (g * L + k) * FEAT
            pltpu.make_async_copy(rowbuf.at[pl.ds(slot, FEAT)],
                                  obxf.at[pl.ds((base + r) * FEAT, FEAT)],
                                  sem_s).wait()
        return 0
      lax.fori_loop(0, (m + L - 1) // L, _blk, 0)

    pl.run_scoped(_phase3, pltpu.VMEM((CAPL * FEAT,), jnp.float32))


_mesh = plsc.VectorSubcoreMesh(core_axis_name="c", subcore_axis_name="s",
                               num_cores=NC, num_subcores=NS)

_sc_call = pl.kernel(
    _body,
    out_type=(jax.ShapeDtypeStruct((CAP * FEAT,), jnp.float32),
              jax.ShapeDtypeStruct((CAP,), jnp.int32)),
    mesh=_mesh,
    compiler_params=pltpu.CompilerParams(needs_layout_passes=False),
    scratch_types=[
        pltpu.VMEM((LISTN,), jnp.int32),
        pltpu.VMEM((LISTN,), jnp.int32),
        pltpu.VMEM((LISTN,), jnp.int32),
        pltpu.VMEM((LISTN,), jnp.int32),
        pltpu.VMEM((LISTN,), jnp.int32),
        pltpu.VMEM((L,), jnp.int32),
        pltpu.SemaphoreType.DMA,
        pltpu.SemaphoreType.DMA,
        pltpu.SemaphoreType.DMA,
        pltpu.SemaphoreType.DMA,
        pltpu.SemaphoreType.DMA,
    ],
)


def kernel(bx, by, x, y, idx):
  obxf, oby = _sc_call(bx.reshape(CAP * FEAT), by, x.reshape(B * FEAT), y, idx)
  return obxf.reshape(CAP, FEAT), oby
